# baseline (speedup 1.0000x reference)
import functools

import jax
import jax.numpy as jnp
from jax import lax
from jax.experimental import pallas as pl
from jax.experimental.pallas import tpu as pltpu
from jax.experimental.pallas import tpu_sc as plsc

VOCAB = 8192
D = 8192
BATCH = 4096

NC = 2
NS = 16
NW = NC * NS
B_PER_W = BATCH // NW
CHUNK = 4
NBUF = 2
N_CHUNKS = B_PER_W // CHUNK

_mesh = plsc.VectorSubcoreMesh(core_axis_name="c", subcore_axis_name="s")


@functools.partial(
    pl.kernel,
    mesh=_mesh,
    out_type=jax.ShapeDtypeStruct((NW, N_CHUNKS, CHUNK, D), jnp.float32),
    scratch_types=[
        pltpu.VMEM((N_CHUNKS, CHUNK), jnp.int32),
        pltpu.VMEM((CHUNK, D), jnp.float32),
        pltpu.VMEM((CHUNK, D), jnp.float32),
        pltpu.SemaphoreType.DMA,
        pltpu.SemaphoreType.DMA,
    ],
)
def _gather_rows(x_hbm, table_hbm, out_hbm, idx_v, buf0, buf1, sem0, sem1):
    wid = lax.axis_index("s") * NC + lax.axis_index("c")
    bufs = (buf0, buf1)
    sems = (sem0, sem1)

    pltpu.sync_copy(x_hbm.at[wid], idx_v)

    def start_gather(c, b):
        pltpu.make_async_copy(table_hbm.at[idx_v.at[c]], bufs[b], sems[b]).start()

    def wait_gather(b):
        pltpu.make_async_copy(table_hbm.at[idx_v.at[0]], bufs[b], sems[b]).wait()

    start_gather(0, 0)

    def outer(g, carry):
        for b in range(NBUF):
            c = g * NBUF + b
            wait_gather(b)

            @pl.when(c + 1 < N_CHUNKS)
            def _prefetch():
                start_gather(c + 1, (b + 1) % NBUF)

            pltpu.sync_copy(bufs[b], out_hbm.at[wid, c])
        return carry

    lax.fori_loop(0, N_CHUNKS // NBUF, outer, 0)


def kernel(x, probs):
    x_chunked = x.astype(jnp.int32).reshape(NW, N_CHUNKS, CHUNK)
    out = _gather_rows(x_chunked, probs)
    return out.reshape(BATCH, D)


# speedup vs baseline: 2.3123x; 2.3123x over previous
import functools

import jax
import jax.numpy as jnp
from jax import lax
from jax.experimental import pallas as pl
from jax.experimental.pallas import tpu as pltpu
from jax.experimental.pallas import tpu_sc as plsc

VOCAB = 8192
D = 8192
BATCH = 4096

NC = 2
NS = 16
NW = NC * NS
B_PER_W = BATCH // NW
CHUNK = 4
NBUF = 2
N_CHUNKS = B_PER_W // CHUNK

_mesh = plsc.VectorSubcoreMesh(core_axis_name="c", subcore_axis_name="s")


@functools.partial(
    pl.kernel,
    mesh=_mesh,
    out_type=jax.ShapeDtypeStruct((BATCH, D), jnp.float32),
    scratch_types=[
        pltpu.VMEM((N_CHUNKS, CHUNK), jnp.int32),
        pltpu.VMEM((CHUNK, D), jnp.float32),
        pltpu.VMEM((CHUNK, D), jnp.float32),
        pltpu.SemaphoreType.DMA,
        pltpu.SemaphoreType.DMA,
    ],
)
def _gather_rows(x_hbm, table_hbm, out_hbm, idx_v, buf0, buf1, sem0, sem1):
    wid = lax.axis_index("s") * NC + lax.axis_index("c")
    bufs = (buf0, buf1)
    sems = (sem0, sem1)

    pltpu.sync_copy(x_hbm.at[wid], idx_v)

    def start_gather(c, b):
        pltpu.make_async_copy(table_hbm.at[idx_v.at[c]], bufs[b], sems[b]).start()

    def wait_gather(b):
        pltpu.make_async_copy(table_hbm.at[idx_v.at[0]], bufs[b], sems[b]).wait()

    start_gather(0, 0)

    def outer(g, carry):
        for b in range(NBUF):
            c = g * NBUF + b
            wait_gather(b)

            @pl.when(c + 1 < N_CHUNKS)
            def _prefetch():
                start_gather(c + 1, (b + 1) % NBUF)

            pltpu.sync_copy(bufs[b], out_hbm.at[pl.ds(wid * B_PER_W + c * CHUNK, CHUNK)])
        return carry

    lax.fori_loop(0, N_CHUNKS // NBUF, outer, 0)


def kernel(x, probs):
    x_chunked = x.astype(jnp.int32).reshape(NW, N_CHUNKS, CHUNK)
    return _gather_rows(x_chunked, probs)
